# baseline (speedup 1.0000x reference)
import functools

import numpy as np

import jax
import jax.numpy as jnp
from jax import lax
from jax.experimental import pallas as pl
from jax.experimental.pallas import tpu as pltpu
from jax.experimental.pallas import tpu_sc as plsc

B = 16384
D = 128
UF = 64
CD = 128
H = 20
NC = 2
NS = 16
NW = NC * NS
P = 2
CHUNK = 128
SH = B // P // NW
M_SC = 25
SCS = M_SC * CHUNK // H
VS = 6
NV = (SH - SCS) // VS


def _sc_gather_body(s, n_idc,
                    tags_s, tags_v, uids3, iids3, pos3, user_table, tag_table,
                    item_table, uid_out, tsum_out, iid_out,
                    tidx_v, vidx_v, pos_v, idx_v,
                    rows0, rows1, rows2, rows3, rows4, acc_v, acc_sh,
                    sem0, sem1, sem2, sem3, sem4,
                    ssem0, ssem1, ssem2, ssem3):
    cid = lax.axis_index("c")
    sid = lax.axis_index("s")
    wid = sid * NC + cid
    base = wid * s
    slab = sid * SCS
    bufs = (rows0, rows1, rows2, rows3)
    sems = (sem0, sem1, sem2, sem3)
    ssems = (ssem0, ssem1, ssem2, ssem3)

    def _zero_row(i, carry):
        for cc in range(D // 16):
            rows0[i, pl.ds(cc * 16, 16)] = jnp.zeros((16,), jnp.float32)
        return carry
    lax.fori_loop(0, CHUNK, _zero_row, 0)
    pltpu.sync_copy(rows0, acc_sh.at[pl.ds(slab, CHUNK)])
    pltpu.sync_copy(rows0.at[pl.ds(0, SCS - CHUNK)],
                    acc_sh.at[pl.ds(slab + CHUNK, SCS - CHUNK)])

    pltpu.sync_copy(uids3.at[wid], idx_v.at[pl.ds(0, n_idc)])
    pltpu.sync_copy(iids3.at[wid], idx_v.at[pl.ds(n_idc, n_idc)])
    gcps, wcps = [], []
    for j in range(2 * n_idc):
        table = user_table if j < n_idc else item_table
        gcps.append(pltpu.async_copy(table.at[idx_v.at[j]], bufs[j], sems[j]))
    for j in range(2 * n_idc):
        out = uid_out if j < n_idc else iid_out
        c = j % n_idc
        gcps[j].wait()
        wcps.append(pltpu.async_copy(bufs[j],
                                     out.at[pl.ds(base + c * CHUNK, CHUNK)],
                                     ssems[j]))
    for w in wcps:
        w.wait()

    pltpu.sync_copy(tags_s.at[wid], tidx_v)
    pltpu.sync_copy(tags_v.at[wid], vidx_v)
    pltpu.sync_copy(pos3.at[sid], pos_v)

    def _vacc(buf, j, r):
        for cc in range(D // 16):
            a = buf[H * j, pl.ds(cc * 16, 16)]
            for jj in range(1, H):
                a = a + buf[H * j + jj, pl.ds(cc * 16, 16)]
            acc_v[r, pl.ds(cc * 16, 16)] = a

    for t in range(3):
        pltpu.async_copy(tag_table.at[tidx_v.at[t]], bufs[t], sems[t])
    pltpu.async_copy(tag_table.at[vidx_v.at[0]], rows3, sem3)
    pltpu.async_copy(tag_table.at[vidx_v.at[1]], rows4, sem4)

    def _iter(i, carry):
        vA = 2 * i
        vB = 2 * i + 1
        for t in range(3):
            c = 3 * i + t
            pltpu.make_async_copy(tag_table.at[tidx_v.at[c]], bufs[t],
                                  sems[t]).wait()
            scp = pltpu.async_copy(bufs[t], acc_sh.at[pos_v.at[c]], ssems[t],
                                   add=True)
            if t == 0:
                pltpu.make_async_copy(tag_table.at[vidx_v.at[vA]], rows3,
                                      sem3).wait()
                for j in range(4):
                    _vacc(rows3, j, 2 * VS * i + j)
            elif t == 1:
                for j in range(4, VS):
                    _vacc(rows3, j, 2 * VS * i + j)

                @pl.when(vA + 2 < NV)
                def _():
                    pltpu.async_copy(tag_table.at[vidx_v.at[vA + 2]], rows3,
                                     sem3)

                pltpu.make_async_copy(tag_table.at[vidx_v.at[vB]], rows4,
                                      sem4).wait()
                for j in range(2):
                    _vacc(rows4, j, 2 * VS * i + VS + j)
            else:
                for j in range(2, VS):
                    _vacc(rows4, j, 2 * VS * i + VS + j)

                @pl.when(vB + 2 < NV)
                def _():
                    pltpu.async_copy(tag_table.at[vidx_v.at[vB + 2]], rows4,
                                     sem4)
            scp.wait()

            @pl.when(c + 3 < M_SC)
            def _():
                pltpu.async_copy(tag_table.at[tidx_v.at[c + 3]], bufs[t],
                                 sems[t])
        return carry
    lax.fori_loop(0, NV // 2, _iter, 0)

    c = M_SC - 1
    t = c % 3
    pltpu.make_async_copy(tag_table.at[tidx_v.at[c]], bufs[t], sems[t]).wait()
    pltpu.async_copy(bufs[t], acc_sh.at[pos_v.at[c]], ssems[t],
                     add=True).wait()

    pltpu.sync_copy(acc_sh.at[pl.ds(slab, SCS)],
                    tsum_out.at[pl.ds(base, SCS)])
    pltpu.sync_copy(acc_v, tsum_out.at[pl.ds(base + SCS, s - SCS)])


def _sc_gather(user_ids, user_tags_idx, item_ids, user_table, tag_table,
               item_table):
    n = user_ids.shape[0]
    s = n // NW
    n_idc = s // CHUNK
    flat = user_tags_idx.reshape(NW, s * H)
    tags_s = flat[:, :M_SC * CHUNK].reshape(NW, M_SC, CHUNK)
    tv = flat[:, M_SC * CHUNK:].reshape(NW, NV, VS * H)
    tags_v = jnp.concatenate([tv, tv[:, :, :CHUNK - VS * H]], axis=-1)
    uids3 = user_ids.reshape(NW, n_idc, CHUNK)
    iids3 = item_ids.reshape(NW, n_idc, CHUNK)
    pos3 = jnp.asarray((np.arange(NS, dtype=np.int32)[:, None] * SCS
                        + np.repeat(np.arange(SCS, dtype=np.int32), H)[None]
                        ).reshape(NS, M_SC, CHUNK))
    mesh = plsc.VectorSubcoreMesh(core_axis_name="c", subcore_axis_name="s")
    fn = pl.kernel(
        functools.partial(_sc_gather_body, s, n_idc),
        out_type=[jax.ShapeDtypeStruct((n, D), jnp.float32) for _ in range(3)],
        mesh=mesh,
        scratch_types=[
            pltpu.VMEM((M_SC, CHUNK), jnp.int32),
            pltpu.VMEM((NV, CHUNK), jnp.int32),
            pltpu.VMEM((M_SC, CHUNK), jnp.int32),
            pltpu.VMEM((2 * n_idc, CHUNK), jnp.int32),
            pltpu.VMEM((CHUNK, D), jnp.float32),
            pltpu.VMEM((CHUNK, D), jnp.float32),
            pltpu.VMEM((CHUNK, D), jnp.float32),
            pltpu.VMEM((CHUNK, D), jnp.float32),
            pltpu.VMEM((CHUNK, D), jnp.float32),
            pltpu.VMEM((s - SCS, D), jnp.float32),
            pltpu.VMEM_SHARED((NS * SCS, D), jnp.float32),
            pltpu.SemaphoreType.DMA,
            pltpu.SemaphoreType.DMA,
            pltpu.SemaphoreType.DMA,
            pltpu.SemaphoreType.DMA,
            pltpu.SemaphoreType.DMA,
            pltpu.SemaphoreType.DMA,
            pltpu.SemaphoreType.DMA,
            pltpu.SemaphoreType.DMA,
            pltpu.SemaphoreType.DMA,
        ],
    )
    return fn(tags_s, tags_v, uids3, iids3, pos3, user_table, tag_table,
              item_table)


def _dot(a, w):
    return lax.dot_general(a.astype(jnp.bfloat16), w.astype(jnp.bfloat16),
                           (((1,), (0,)), ((), ())),
                           preferred_element_type=jnp.float32)


def _ln_aff(x, g, b, eps=1e-5):
    m = jnp.mean(x, axis=-1, keepdims=True)
    v = jnp.mean((x - m) ** 2, axis=-1, keepdims=True)
    return (x - m) * lax.rsqrt(v + eps) * g + b


def _tc_body(uid, tsum, iid, cf, tp_W, tp_b, tp_g, tp_beta, uf_W, uf_b, uf_g,
             uf_beta, cf_W, cf_b, cf_g, cf_beta, if_W, if_b, if_g, if_beta,
             m1_W, m1_b, m2_W, m2_b, m3_W, m3_b, p_W, p_b, out_ref):
    tag = tsum[...] * (1.0 / H)
    t = _ln_aff(jax.nn.relu(_dot(tag, tp_W[...]) + tp_b[...]),
                tp_g[...], tp_beta[...])
    ufW = uf_W[...]
    ue = _ln_aff(jax.nn.relu(_dot(uid[...], ufW[:D]) + _dot(t, ufW[D:])
                             + uf_b[...]), uf_g[...], uf_beta[...])
    ce = _ln_aff(jax.nn.relu(_dot(cf[...], cf_W[...]) + cf_b[...]),
                 cf_g[...], cf_beta[...])
    ifW = if_W[...]
    ie = _ln_aff(jax.nn.relu(_dot(iid[...], ifW[:D]) + _dot(ce, ifW[D:])
                             + if_b[...]), if_g[...], if_beta[...])
    m1W = m1_W[...]
    h = jax.nn.relu(_dot(ue, m1W[:UF]) + _dot(ie, m1W[UF:]) + m1_b[...])
    h = jax.nn.relu(_dot(h, m2_W[...]) + m2_b[...])
    h = jax.nn.relu(_dot(h, m3_W[...]) + m3_b[...])
    logit = jnp.sum(h * p_W[...], axis=1) + p_b[0, 0]
    out_ref[...] = jax.nn.sigmoid(logit)


def kernel(user_ids, user_tags_idx, item_ids, content_features, user_table,
           tag_table, item_table, tp_W, tp_b, tp_g, tp_beta, uf_W, uf_b, uf_g,
           uf_beta, cf_W, cf_b, cf_g, cf_beta, if_W, if_b, if_g, if_beta,
           m1_W, m1_b, m2_W, m2_b, m3_W, m3_b, p_W, p_b):
    weights = [tp_W.T, tp_b.reshape(1, -1), tp_g.reshape(1, -1),
               tp_beta.reshape(1, -1), uf_W.T, uf_b.reshape(1, -1),
               uf_g.reshape(1, -1), uf_beta.reshape(1, -1), cf_W.T,
               cf_b.reshape(1, -1), cf_g.reshape(1, -1),
               cf_beta.reshape(1, -1), if_W.T, if_b.reshape(1, -1),
               if_g.reshape(1, -1), if_beta.reshape(1, -1), m1_W.T,
               m1_b.reshape(1, -1), m2_W.T, m2_b.reshape(1, -1), m3_W.T,
               m3_b.reshape(1, -1), p_W, p_b.reshape(1, -1)]

    BH = B // P
    BM = 2048
    row = pl.BlockSpec((BM, D), lambda i: (i, 0))
    full = lambda a: pl.BlockSpec(a.shape, lambda i: tuple(0 for _ in a.shape))
    dense = pl.pallas_call(
        _tc_body,
        grid=(BH // BM,),
        in_specs=[row, row, row, row] + [full(w) for w in weights],
        out_specs=pl.BlockSpec((BM,), lambda i: (i,)),
        out_shape=jax.ShapeDtypeStruct((BH,), jnp.float32),
    )

    outs = []
    for p in range(P):
        sl = slice(p * BH, (p + 1) * BH)
        uid, tsum, iid = _sc_gather(user_ids[sl], user_tags_idx[sl],
                                    item_ids[sl], user_table, tag_table,
                                    item_table)
        outs.append(dense(uid, tsum, iid, content_features[sl], *weights))
    return jnp.concatenate(outs)


# speedup vs baseline: 1.2335x; 1.2335x over previous
import functools

import numpy as np

import jax
import jax.numpy as jnp
from jax import lax
from jax.experimental import pallas as pl
from jax.experimental.pallas import tpu as pltpu
from jax.experimental.pallas import tpu_sc as plsc

B = 16384
D = 128
UF = 64
CD = 128
H = 20
NC = 2
NS = 16
NW = NC * NS
P = 2
CHUNK = 128
SH = B // P // NW
M_SC = 25
SCS = M_SC * CHUNK // H
VS = 6
NV = (SH - SCS) // VS


def _sc_gather_body(s, n_idc,
                    tags_s, tags_v, uids3, iids3, pos3, user_table, tag_table,
                    item_table, uid_out, tsum_out, iid_out,
                    tidx_v, vidx_v, pos_v, idx_v,
                    rows0, rows1, rows2, rows3, rows4, acc_v, acc_sh,
                    sem0, sem1, sem2, sem3, sem4,
                    ssem0, ssem1, ssem2, ssem3):
    cid = lax.axis_index("c")
    sid = lax.axis_index("s")
    wid = sid * NC + cid
    base = wid * s
    slab = sid * SCS
    bufs = (rows0, rows1, rows2, rows3)
    sems = (sem0, sem1, sem2, sem3)
    ssems = (ssem0, ssem1, ssem2, ssem3)

    def _zero_row(i, carry):
        for cc in range(D // 16):
            rows0[i, pl.ds(cc * 16, 16)] = jnp.zeros((16,), jnp.float32)
        return carry
    lax.fori_loop(0, CHUNK, _zero_row, 0)
    pltpu.sync_copy(rows0, acc_sh.at[pl.ds(slab, CHUNK)])
    pltpu.sync_copy(rows0.at[pl.ds(0, SCS - CHUNK)],
                    acc_sh.at[pl.ds(slab + CHUNK, SCS - CHUNK)])

    pltpu.sync_copy(uids3.at[wid], idx_v.at[pl.ds(0, n_idc)])
    pltpu.sync_copy(iids3.at[wid], idx_v.at[pl.ds(n_idc, n_idc)])
    gcps, wcps = [], []
    for j in range(2 * n_idc):
        table = user_table if j < n_idc else item_table
        gcps.append(pltpu.async_copy(table.at[idx_v.at[j]], bufs[j], sems[j]))
    for j in range(2 * n_idc):
        out = uid_out if j < n_idc else iid_out
        c = j % n_idc
        gcps[j].wait()
        wcps.append(pltpu.async_copy(bufs[j],
                                     out.at[pl.ds(base + c * CHUNK, CHUNK)],
                                     ssems[j]))
    for w in wcps:
        w.wait()

    pltpu.sync_copy(tags_s.at[wid], tidx_v)
    pltpu.sync_copy(tags_v.at[wid], vidx_v)
    pltpu.sync_copy(pos3.at[sid], pos_v)

    def _vacc(buf, j, r):
        def _cc(cc, carry):
            a = buf[H * j, pl.ds(cc * 16, 16)]
            for jj in range(1, H):
                a = a + buf[H * j + jj, pl.ds(cc * 16, 16)]
            acc_v[r, pl.ds(cc * 16, 16)] = a
            return carry
        lax.fori_loop(0, D // 16, _cc, 0)

    for t in range(3):
        pltpu.async_copy(tag_table.at[tidx_v.at[t]], bufs[t], sems[t])
    pltpu.async_copy(tag_table.at[vidx_v.at[0]], rows3, sem3)
    pltpu.async_copy(tag_table.at[vidx_v.at[1]], rows4, sem4)

    def _iter(i, carry):
        vA = 2 * i
        vB = 2 * i + 1
        for t in range(3):
            c = 3 * i + t
            pltpu.make_async_copy(tag_table.at[tidx_v.at[c]], bufs[t],
                                  sems[t]).wait()
            scp = pltpu.async_copy(bufs[t], acc_sh.at[pos_v.at[c]], ssems[t],
                                   add=True)
            if t == 0:
                pltpu.make_async_copy(tag_table.at[vidx_v.at[vA]], rows3,
                                      sem3).wait()
                for j in range(4):
                    _vacc(rows3, j, 2 * VS * i + j)
            elif t == 1:
                for j in range(4, VS):
                    _vacc(rows3, j, 2 * VS * i + j)

                @pl.when(vA + 2 < NV)
                def _():
                    pltpu.async_copy(tag_table.at[vidx_v.at[vA + 2]], rows3,
                                     sem3)

                pltpu.make_async_copy(tag_table.at[vidx_v.at[vB]], rows4,
                                      sem4).wait()
                for j in range(2):
                    _vacc(rows4, j, 2 * VS * i + VS + j)
            else:
                for j in range(2, VS):
                    _vacc(rows4, j, 2 * VS * i + VS + j)

                @pl.when(vB + 2 < NV)
                def _():
                    pltpu.async_copy(tag_table.at[vidx_v.at[vB + 2]], rows4,
                                     sem4)
            scp.wait()

            @pl.when(c + 3 < M_SC)
            def _():
                pltpu.async_copy(tag_table.at[tidx_v.at[c + 3]], bufs[t],
                                 sems[t])
        return carry
    lax.fori_loop(0, NV // 2, _iter, 0)

    c = M_SC - 1
    t = c % 3
    pltpu.make_async_copy(tag_table.at[tidx_v.at[c]], bufs[t], sems[t]).wait()
    pltpu.async_copy(bufs[t], acc_sh.at[pos_v.at[c]], ssems[t],
                     add=True).wait()

    pltpu.sync_copy(acc_sh.at[pl.ds(slab, SCS)],
                    tsum_out.at[pl.ds(base, SCS)])
    pltpu.sync_copy(acc_v, tsum_out.at[pl.ds(base + SCS, s - SCS)])


def _sc_gather(user_ids, user_tags_idx, item_ids, user_table, tag_table,
               item_table):
    n = user_ids.shape[0]
    s = n // NW
    n_idc = s // CHUNK
    flat = user_tags_idx.reshape(NW, s * H)
    tags_s = flat[:, :M_SC * CHUNK].reshape(NW, M_SC, CHUNK)
    tv = flat[:, M_SC * CHUNK:].reshape(NW, NV, VS * H)
    tags_v = jnp.concatenate([tv, tv[:, :, :CHUNK - VS * H]], axis=-1)
    uids3 = user_ids.reshape(NW, n_idc, CHUNK)
    iids3 = item_ids.reshape(NW, n_idc, CHUNK)
    pos3 = jnp.asarray((np.arange(NS, dtype=np.int32)[:, None] * SCS
                        + np.repeat(np.arange(SCS, dtype=np.int32), H)[None]
                        ).reshape(NS, M_SC, CHUNK))
    mesh = plsc.VectorSubcoreMesh(core_axis_name="c", subcore_axis_name="s")
    fn = pl.kernel(
        functools.partial(_sc_gather_body, s, n_idc),
        out_type=[jax.ShapeDtypeStruct((n, D), jnp.float32) for _ in range(3)],
        mesh=mesh,
        scratch_types=[
            pltpu.VMEM((M_SC, CHUNK), jnp.int32),
            pltpu.VMEM((NV, CHUNK), jnp.int32),
            pltpu.VMEM((M_SC, CHUNK), jnp.int32),
            pltpu.VMEM((2 * n_idc, CHUNK), jnp.int32),
            pltpu.VMEM((CHUNK, D), jnp.float32),
            pltpu.VMEM((CHUNK, D), jnp.float32),
            pltpu.VMEM((CHUNK, D), jnp.float32),
            pltpu.VMEM((CHUNK, D), jnp.float32),
            pltpu.VMEM((CHUNK, D), jnp.float32),
            pltpu.VMEM((s - SCS, D), jnp.float32),
            pltpu.VMEM_SHARED((NS * SCS, D), jnp.float32),
            pltpu.SemaphoreType.DMA,
            pltpu.SemaphoreType.DMA,
            pltpu.SemaphoreType.DMA,
            pltpu.SemaphoreType.DMA,
            pltpu.SemaphoreType.DMA,
            pltpu.SemaphoreType.DMA,
            pltpu.SemaphoreType.DMA,
            pltpu.SemaphoreType.DMA,
            pltpu.SemaphoreType.DMA,
        ],
    )
    return fn(tags_s, tags_v, uids3, iids3, pos3, user_table, tag_table,
              item_table)


def _dot(a, w):
    return lax.dot_general(a.astype(jnp.bfloat16), w.astype(jnp.bfloat16),
                           (((1,), (0,)), ((), ())),
                           preferred_element_type=jnp.float32)


def _ln_aff(x, g, b, eps=1e-5):
    m = jnp.mean(x, axis=-1, keepdims=True)
    v = jnp.mean((x - m) ** 2, axis=-1, keepdims=True)
    return (x - m) * lax.rsqrt(v + eps) * g + b


def _tc_body(uid, tsum, iid, cf, tp_W, tp_b, tp_g, tp_beta, uf_W, uf_b, uf_g,
             uf_beta, cf_W, cf_b, cf_g, cf_beta, if_W, if_b, if_g, if_beta,
             m1_W, m1_b, m2_W, m2_b, m3_W, m3_b, p_W, p_b, out_ref):
    tag = tsum[...] * (1.0 / H)
    t = _ln_aff(jax.nn.relu(_dot(tag, tp_W[...]) + tp_b[...]),
                tp_g[...], tp_beta[...])
    ufW = uf_W[...]
    ue = _ln_aff(jax.nn.relu(_dot(uid[...], ufW[:D]) + _dot(t, ufW[D:])
                             + uf_b[...]), uf_g[...], uf_beta[...])
    ce = _ln_aff(jax.nn.relu(_dot(cf[...], cf_W[...]) + cf_b[...]),
                 cf_g[...], cf_beta[...])
    ifW = if_W[...]
    ie = _ln_aff(jax.nn.relu(_dot(iid[...], ifW[:D]) + _dot(ce, ifW[D:])
                             + if_b[...]), if_g[...], if_beta[...])
    m1W = m1_W[...]
    h = jax.nn.relu(_dot(ue, m1W[:UF]) + _dot(ie, m1W[UF:]) + m1_b[...])
    h = jax.nn.relu(_dot(h, m2_W[...]) + m2_b[...])
    h = jax.nn.relu(_dot(h, m3_W[...]) + m3_b[...])
    logit = jnp.sum(h * p_W[...], axis=1) + p_b[0, 0]
    out_ref[...] = jax.nn.sigmoid(logit)


def kernel(user_ids, user_tags_idx, item_ids, content_features, user_table,
           tag_table, item_table, tp_W, tp_b, tp_g, tp_beta, uf_W, uf_b, uf_g,
           uf_beta, cf_W, cf_b, cf_g, cf_beta, if_W, if_b, if_g, if_beta,
           m1_W, m1_b, m2_W, m2_b, m3_W, m3_b, p_W, p_b):
    weights = [tp_W.T, tp_b.reshape(1, -1), tp_g.reshape(1, -1),
               tp_beta.reshape(1, -1), uf_W.T, uf_b.reshape(1, -1),
               uf_g.reshape(1, -1), uf_beta.reshape(1, -1), cf_W.T,
               cf_b.reshape(1, -1), cf_g.reshape(1, -1),
               cf_beta.reshape(1, -1), if_W.T, if_b.reshape(1, -1),
               if_g.reshape(1, -1), if_beta.reshape(1, -1), m1_W.T,
               m1_b.reshape(1, -1), m2_W.T, m2_b.reshape(1, -1), m3_W.T,
               m3_b.reshape(1, -1), p_W, p_b.reshape(1, -1)]

    BH = B // P
    BM = 2048
    row = pl.BlockSpec((BM, D), lambda i: (i, 0))
    full = lambda a: pl.BlockSpec(a.shape, lambda i: tuple(0 for _ in a.shape))
    dense = pl.pallas_call(
        _tc_body,
        grid=(BH // BM,),
        in_specs=[row, row, row, row] + [full(w) for w in weights],
        out_specs=pl.BlockSpec((BM,), lambda i: (i,)),
        out_shape=jax.ShapeDtypeStruct((BH,), jnp.float32),
    )

    outs = []
    for p in range(P):
        sl = slice(p * BH, (p + 1) * BH)
        uid, tsum, iid = _sc_gather(user_ids[sl], user_tags_idx[sl],
                                    item_ids[sl], user_table, tag_table,
                                    item_table)
        outs.append(dense(uid, tsum, iid, content_features[sl], *weights))
    return jnp.concatenate(outs)
